# baseline (speedup 1.0000x reference)
import functools
import math

import jax
import jax.numpy as jnp
from jax import lax
from jax.experimental import pallas as pl
from jax.experimental.pallas import tpu as pltpu
from jax.experimental.pallas import tpu_sc as plsc

_NUM_CORES = 2
_NUM_SUBCORES = 16
_NUM_WORKERS = _NUM_CORES * _NUM_SUBCORES
_LANES = 16

_CHUNK = 16
_NBUF = 4


def _build_sc_kernel(vocab, d, n_rows, seq_len):
    segs = d // _LANES
    rows_per_w = n_rows // _NUM_WORKERS
    n_chunks = rows_per_w // _CHUNK
    scale = float(math.sqrt(float(d)))

    mesh = plsc.VectorSubcoreMesh(core_axis_name="c", subcore_axis_name="s")

    @functools.partial(
        pl.kernel,
        mesh=mesh,
        out_type=jax.ShapeDtypeStruct((n_rows, d), jnp.float32),
        scratch_types=[
            pltpu.VMEM((n_chunks, _CHUNK), jnp.int32),
            pltpu.VMEM((d,), jnp.float32),
            pltpu.VMEM((_NBUF, _CHUNK, d), jnp.float32),
            pltpu.SemaphoreType.DMA((_NBUF,)),
            pltpu.SemaphoreType.DMA((_NBUF,)),
        ],
    )
    def sc_kernel(table_hbm, idx_hbm, invf_hbm, out_hbm, idx_v, invf_v,
                  rows_v, gsem, osem):
        wid = lax.axis_index("s") * _NUM_CORES + lax.axis_index("c")
        base = wid * rows_per_w

        pltpu.sync_copy(idx_hbm.at[wid], idx_v)
        pltpu.sync_copy(invf_hbm, invf_v)

        def mk_gather(c, buf):
            return pltpu.make_async_copy(
                table_hbm.at[idx_v.at[c]], rows_v.at[buf], gsem.at[buf])

        def mk_out(c, buf):
            return pltpu.make_async_copy(
                rows_v.at[buf],
                out_hbm.at[pl.ds(base + c * _CHUNK, _CHUNK)],
                osem.at[buf])

        def compute(c, buf):
            posbase = (base + c * _CHUNK) % seq_len
            buf_ref = rows_v.at[buf]

            @pl.loop(0, segs)
            def _(s):
                col = s * _LANES
                invf = invf_v[pl.ds(col, _LANES)]

                @plsc.parallel_loop(0, _CHUNK, unroll=8)
                def _(r):
                    posf = (posbase + r).astype(jnp.float32)
                    g = buf_ref[r, pl.ds(col, _LANES)]
                    buf_ref[r, pl.ds(col, _LANES)] = g * scale + posf * invf

        out_copies = [None] * n_chunks
        gather_copies = [None] * n_chunks
        for c in (0, 1):
            gather_copies[c] = mk_gather(c, c % _NBUF)
            gather_copies[c].start()
        for c in range(n_chunks):
            buf = c % _NBUF
            if c + 2 < n_chunks:
                if c - 2 >= 0:
                    out_copies[c - 2].wait()
                gather_copies[c + 2] = mk_gather(c + 2, (c + 2) % _NBUF)
                gather_copies[c + 2].start()
            gather_copies[c].wait()
            compute(c, buf)
            out_copies[c] = mk_out(c, buf)
            out_copies[c].start()
        for c in range(max(0, n_chunks - 4), n_chunks):
            out_copies[c].wait()

    return sc_kernel


def kernel(sequence, embedding_table):
    b, s = sequence.shape
    vocab, d = embedding_table.shape
    n_rows = b * s
    idx = sequence.reshape(_NUM_WORKERS, n_rows // (_NUM_WORKERS * _CHUNK),
                           _CHUNK)
    j = jnp.arange(d, dtype=jnp.float32)
    inv_freq = jnp.power(jnp.float32(10000.0), -(j - jnp.mod(j, 2.0)) / d)
    sc = _build_sc_kernel(vocab, d, n_rows, s)
    out = sc(embedding_table, idx, inv_freq)
    return out.reshape(b, s, d)


# speedup vs baseline: 1.1750x; 1.1750x over previous
import functools
import math

import jax
import jax.numpy as jnp
from jax import lax
from jax.experimental import pallas as pl
from jax.experimental.pallas import tpu as pltpu
from jax.experimental.pallas import tpu_sc as plsc

_NUM_CORES = 2
_NUM_SUBCORES = 16
_NUM_WORKERS = _NUM_CORES * _NUM_SUBCORES
_LANES = 16

_CHUNK = 32
_NBUF = 3


def _build_sc_kernel(vocab, d, n_rows, seq_len):
    segs = d // _LANES
    rows_per_w = n_rows // _NUM_WORKERS
    n_chunks = rows_per_w // _CHUNK
    scale = float(math.sqrt(float(d)))

    mesh = plsc.VectorSubcoreMesh(core_axis_name="c", subcore_axis_name="s")

    @functools.partial(
        pl.kernel,
        mesh=mesh,
        out_type=jax.ShapeDtypeStruct((n_rows, d), jnp.float32),
        scratch_types=[
            pltpu.VMEM((n_chunks, _CHUNK), jnp.int32),
            pltpu.VMEM((d,), jnp.float32),
            pltpu.VMEM((_NBUF, _CHUNK, d), jnp.float32),
            pltpu.SemaphoreType.DMA((_NBUF,)),
            pltpu.SemaphoreType.DMA((_NBUF,)),
        ],
    )
    def sc_kernel(table_hbm, idx_hbm, invf_hbm, out_hbm, idx_v, invf_v,
                  rows_v, gsem, osem):
        wid = lax.axis_index("s") * _NUM_CORES + lax.axis_index("c")
        base = wid * rows_per_w

        pltpu.sync_copy(idx_hbm.at[wid], idx_v)
        pltpu.sync_copy(invf_hbm, invf_v)

        def mk_gather(c, buf):
            return pltpu.make_async_copy(
                table_hbm.at[idx_v.at[c]], rows_v.at[buf], gsem.at[buf])

        def mk_out(c, buf):
            return pltpu.make_async_copy(
                rows_v.at[buf],
                out_hbm.at[pl.ds(base + c * _CHUNK, _CHUNK)],
                osem.at[buf])

        def compute(c, buf):
            posbase = (base + c * _CHUNK) % seq_len
            buf_ref = rows_v.at[buf]

            @pl.loop(0, segs)
            def _(s):
                col = s * _LANES
                invf = invf_v[pl.ds(col, _LANES)]

                @plsc.parallel_loop(0, _CHUNK, unroll=8)
                def _(r):
                    posf = (posbase + r).astype(jnp.float32)
                    g = buf_ref[r, pl.ds(col, _LANES)]
                    buf_ref[r, pl.ds(col, _LANES)] = g * scale + posf * invf

        out_copies = [None] * n_chunks
        gather_copies = [None] * n_chunks
        gather_copies[0] = mk_gather(0, 0)
        gather_copies[0].start()
        for c in range(n_chunks):
            buf = c % _NBUF
            if c + 1 < n_chunks:
                if c - 2 >= 0:
                    out_copies[c - 2].wait()
                gather_copies[c + 1] = mk_gather(c + 1, (c + 1) % _NBUF)
                gather_copies[c + 1].start()
            gather_copies[c].wait()
            compute(c, buf)
            out_copies[c] = mk_out(c, buf)
            out_copies[c].start()
        for c in range(max(0, n_chunks - _NBUF), n_chunks):
            out_copies[c].wait()

    return sc_kernel


def kernel(sequence, embedding_table):
    b, s = sequence.shape
    vocab, d = embedding_table.shape
    n_rows = b * s
    idx = sequence.reshape(_NUM_WORKERS, n_rows // (_NUM_WORKERS * _CHUNK),
                           _CHUNK)
    j = jnp.arange(d, dtype=jnp.float32)
    inv_freq = jnp.power(jnp.float32(10000.0), -(j - jnp.mod(j, 2.0)) / d)
    sc = _build_sc_kernel(vocab, d, n_rows, s)
    out = sc(embedding_table, idx, inv_freq)
    return out.reshape(b, s, d)


# speedup vs baseline: 1.2269x; 1.0442x over previous
import functools
import math

import jax
import jax.numpy as jnp
from jax import lax
from jax.experimental import pallas as pl
from jax.experimental.pallas import tpu as pltpu
from jax.experimental.pallas import tpu_sc as plsc

_NUM_CORES = 2
_NUM_SUBCORES = 16
_NUM_WORKERS = _NUM_CORES * _NUM_SUBCORES
_LANES = 16

_CHUNK = 32
_NBUF = 3


def _build_sc_kernel(vocab, d, n_rows, seq_len):
    segs = d // _LANES
    rows_per_w = n_rows // _NUM_WORKERS
    n_chunks = rows_per_w // _CHUNK
    scale = float(math.sqrt(float(d)))

    mesh = plsc.VectorSubcoreMesh(core_axis_name="c", subcore_axis_name="s")

    @functools.partial(
        pl.kernel,
        mesh=mesh,
        out_type=jax.ShapeDtypeStruct((n_rows, d), jnp.float32),
        scratch_types=[
            pltpu.VMEM((rows_per_w,), jnp.int32),
            pltpu.VMEM((d,), jnp.float32),
            pltpu.VMEM((_NBUF, _CHUNK, d), jnp.float32),
            pltpu.SemaphoreType.DMA((_NBUF,)),
            pltpu.SemaphoreType.DMA((_NBUF,)),
        ],
    )
    def sc_kernel(table_hbm, idx_hbm, invf_hbm, out_hbm, idx_v, invf_v,
                  rows_v, gsem, osem):
        wid = lax.axis_index("s") * _NUM_CORES + lax.axis_index("c")
        base = wid * rows_per_w

        b_row = base // seq_len
        b_col = base % seq_len
        pltpu.sync_copy(idx_hbm.at[b_row, pl.ds(b_col, rows_per_w)], idx_v)

        def mk_gather(c, buf):
            return pltpu.make_async_copy(
                table_hbm.at[idx_v.at[pl.ds(c * _CHUNK, _CHUNK)]],
                rows_v.at[buf], gsem.at[buf])

        def mk_out(c, buf):
            return pltpu.make_async_copy(
                rows_v.at[buf],
                out_hbm.at[pl.ds(base + c * _CHUNK, _CHUNK)],
                osem.at[buf])

        def compute(c, buf):
            posbase = (base + c * _CHUNK) % seq_len
            buf_ref = rows_v.at[buf]

            @pl.loop(0, segs)
            def _(s):
                col = s * _LANES
                invf = invf_v[pl.ds(col, _LANES)]

                @plsc.parallel_loop(0, _CHUNK, unroll=8)
                def _(r):
                    posf = (posbase + r).astype(jnp.float32)
                    g = buf_ref[r, pl.ds(col, _LANES)]
                    buf_ref[r, pl.ds(col, _LANES)] = g * scale + posf * invf

        out_copies = [None] * n_chunks
        gather_copies = [None] * n_chunks
        gather_copies[0] = mk_gather(0, 0)
        gather_copies[0].start()
        pltpu.sync_copy(invf_hbm, invf_v)
        for c in range(n_chunks):
            buf = c % _NBUF
            if c + 1 < n_chunks:
                if c - 2 >= 0:
                    out_copies[c - 2].wait()
                gather_copies[c + 1] = mk_gather(c + 1, (c + 1) % _NBUF)
                gather_copies[c + 1].start()
            gather_copies[c].wait()
            compute(c, buf)
            out_copies[c] = mk_out(c, buf)
            out_copies[c].start()
        for c in range(max(0, n_chunks - _NBUF), n_chunks):
            out_copies[c].wait()

    return sc_kernel


def kernel(sequence, embedding_table):
    b, s = sequence.shape
    vocab, d = embedding_table.shape
    n_rows = b * s
    j = jnp.arange(d, dtype=jnp.float32)
    inv_freq = jnp.power(jnp.float32(10000.0), -(j - jnp.mod(j, 2.0)) / d)
    sc = _build_sc_kernel(vocab, d, n_rows, s)
    out = sc(embedding_table, sequence, inv_freq)
    return out.reshape(b, s, d)
